# baseline (speedup 1.0000x reference)
import functools
import jax
import jax.numpy as jnp
import numpy as np
from jax.experimental import pallas as pl
from jax.experimental.pallas import tpu as pltpu

Q = 1024
K = 100000
DIM = 64
C = 32
K1 = 100
TOPK = 5

KPAD = 102400
BK = 2048
NBLK = KPAD // BK


def _dist_body(q_ref, k_ref, out_ref):
    q = q_ref[...]
    k = k_ref[...]
    q_sq = jnp.sum(q * q, axis=1, keepdims=True)
    k_sq = jnp.sum(k * k, axis=1)
    qk = jax.lax.dot_general(q, k, (((1,), (1,)), ((), ())),
                             preferred_element_type=jnp.float32)
    out_ref[...] = q_sq + k_sq[None, :] - 2.0 * qk


def _dist_matrix(queries, keys_pad):
    return pl.pallas_call(
        _dist_body,
        grid=(NBLK,),
        in_specs=[
            pl.BlockSpec((Q, DIM), lambda i: (0, 0)),
            pl.BlockSpec((BK, DIM), lambda i: (i, 0)),
        ],
        out_specs=pl.BlockSpec((Q, BK), lambda i: (0, i)),
        out_shape=jax.ShapeDtypeStruct((Q, KPAD), jnp.float32),
    )(queries, keys_pad)


def kernel(queries, keys, query_categories, key_categories, complex_weight, WU, aU):
    keys_pad = jnp.pad(keys, ((0, KPAD - K), (0, 0)))
    dist_full = _dist_matrix(queries, keys_pad)
    dist = dist_full[:, :K]

    negD, I = jax.lax.top_k(-dist, K1)
    Dmat = -negD
    simu_category = key_categories[I].astype(bool)
    currentu_category = jnp.broadcast_to(
        query_categories[:, None, :].astype(bool), simu_category.shape)
    remain_category = jnp.logical_and(
        jnp.logical_xor(currentu_category, simu_category), simu_category)
    mask = 1e20 * (1.0 - jnp.any(remain_category, axis=2).astype(jnp.float32))
    Dmat = Dmat + mask
    index = jnp.argsort(Dmat, axis=0)[:, :TOPK]
    line_id = jnp.tile(jnp.arange(index.shape[0]).reshape(-1, 1),
                       (1, index.shape[1])) * index.shape[1]
    new_I = I.reshape(-1)[(line_id + index).reshape(-1)].reshape(index.shape[0], -1)
    simu_embed = keys[new_I]
    x = jnp.fft.rfft(simu_embed, axis=1, norm='ortho')
    weight = complex_weight[..., 0] + 1j * complex_weight[..., 1]
    weight = weight[:, :TOPK // 2 + 1, :]
    x = x * weight
    simu_embed = jnp.fft.irfft(x, n=TOPK, axis=1, norm='ortho')
    batch_size = simu_embed.shape[0]
    Wsimu_embed = (simu_embed.reshape(-1, DIM) @ WU).reshape(batch_size, -1, DIM)
    Wu_embed = jnp.repeat((queries @ WU)[:, None, :], TOPK, axis=1)
    W_embed = jnp.concatenate([Wsimu_embed, Wu_embed], axis=2)
    Wa_embed = jax.nn.leaky_relu(jnp.einsum('bij,j->bi', W_embed, aU),
                                 negative_slope=0.1)
    alpha_embed = jax.nn.softmax(Wa_embed, axis=1)
    anchor_user_embed = jnp.sum(Wsimu_embed * alpha_embed[:, :, None], axis=1)
    return anchor_user_embed


# speedup vs baseline: 9.3761x; 9.3761x over previous
import functools
import jax
import jax.numpy as jnp
import numpy as np
from jax.experimental import pallas as pl
from jax.experimental.pallas import tpu as pltpu

Q = 1024
K = 100000
DIM = 64
C = 32
K1 = 100
TOPK = 5

KPAD = 102400
BK = 2048
NBLK = KPAD // BK
NROW64 = 64
CW = 128


def _distsel_body(q_ref, k_ref, qsq_ref, ksq_ref, cd_ref, ci_ref, d64_ref, d_scr):
    i = pl.program_id(0)
    q = q_ref[...]
    k = k_ref[...]
    lane = jax.lax.broadcasted_iota(jnp.int32, (Q, BK), 1)
    qk = jax.lax.dot_general(q, k, (((1,), (1,)), ((), ())),
                             preferred_element_type=jnp.float32)
    d = qsq_ref[...] + ksq_ref[...] - 2.0 * qk
    d64_ref[...] = d[:NROW64, :]
    d_scr[...] = d

    cd_ref[...] = jnp.full((Q, CW), jnp.inf, dtype=jnp.float32)
    ci_ref[...] = jnp.zeros((Q, CW), dtype=jnp.int32)
    for t in range(TOPK):
        d = d_scr[...]
        m = jnp.min(d, axis=1, keepdims=True)
        is_min = d == m
        a = jnp.min(jnp.where(is_min, lane, jnp.int32(BK)), axis=1,
                    keepdims=True)
        d_scr[...] = jnp.where(lane == a, jnp.float32(jnp.inf), d)
        cd_ref[:, t:t + 1] = m
        ci_ref[:, t:t + 1] = a + i * BK


def _dist_select(queries, keys_pad, q_sq, k_sq_pad):
    return pl.pallas_call(
        _distsel_body,
        grid=(NBLK,),
        in_specs=[
            pl.BlockSpec((Q, DIM), lambda i: (0, 0)),
            pl.BlockSpec((BK, DIM), lambda i: (i, 0)),
            pl.BlockSpec((Q, 1), lambda i: (0, 0)),
            pl.BlockSpec((1, BK), lambda i: (0, i)),
        ],
        out_specs=[
            pl.BlockSpec((Q, CW), lambda i: (0, i)),
            pl.BlockSpec((Q, CW), lambda i: (0, i)),
            pl.BlockSpec((NROW64, BK), lambda i: (0, i)),
        ],
        out_shape=[
            jax.ShapeDtypeStruct((Q, NBLK * CW), jnp.float32),
            jax.ShapeDtypeStruct((Q, NBLK * CW), jnp.int32),
            jax.ShapeDtypeStruct((NROW64, KPAD), jnp.float32),
        ],
        scratch_shapes=[pltpu.VMEM((Q, BK), jnp.float32)],
    )(queries, keys_pad, q_sq, k_sq_pad)


def kernel(queries, keys, query_categories, key_categories, complex_weight, WU, aU):
    keys_pad = jnp.pad(keys, ((0, KPAD - K), (0, 0)))
    q_sq = jnp.sum(queries * queries, axis=1, keepdims=True)
    k_sq = jnp.sum(keys * keys, axis=1)
    k_sq_pad = jnp.pad(k_sq, (0, KPAD - K), constant_values=1e30)[None, :]
    cd, ci, d64 = _dist_select(queries, keys_pad, q_sq, k_sq_pad)

    negv, pos = jax.lax.top_k(-cd, TOPK)
    D5 = -negv
    I5 = jnp.take_along_axis(ci, pos, axis=1)

    negd, I100 = jax.lax.top_k(-d64, K1)

    simu_category = key_categories[I5].astype(bool)
    currentu_category = jnp.broadcast_to(
        query_categories[:, None, :].astype(bool), simu_category.shape)
    remain_category = jnp.logical_and(
        jnp.logical_xor(currentu_category, simu_category), simu_category)
    mask = 1e20 * (1.0 - jnp.any(remain_category, axis=2).astype(jnp.float32))
    Dmat = D5 + mask

    index = jnp.argsort(Dmat, axis=0)
    line_id = jnp.tile(jnp.arange(Q).reshape(-1, 1), (1, TOPK)) * TOPK
    flat_pos = (line_id + index).reshape(-1)
    new_I = I100.reshape(-1)[flat_pos].reshape(Q, TOPK)
    simu_embed = keys[new_I]

    x = jnp.fft.rfft(simu_embed, axis=1, norm='ortho')
    weight = complex_weight[..., 0] + 1j * complex_weight[..., 1]
    weight = weight[:, :TOPK // 2 + 1, :]
    x = x * weight
    simu_embed = jnp.fft.irfft(x, n=TOPK, axis=1, norm='ortho')
    Wsimu_embed = (simu_embed.reshape(-1, DIM) @ WU).reshape(Q, -1, DIM)
    Wu_embed = jnp.repeat((queries @ WU)[:, None, :], TOPK, axis=1)
    W_embed = jnp.concatenate([Wsimu_embed, Wu_embed], axis=2)
    Wa_embed = jax.nn.leaky_relu(jnp.einsum('bij,j->bi', W_embed, aU),
                                 negative_slope=0.1)
    alpha_embed = jax.nn.softmax(Wa_embed, axis=1)
    anchor_user_embed = jnp.sum(Wsimu_embed * alpha_embed[:, :, None], axis=1)
    return anchor_user_embed


# speedup vs baseline: 10.0532x; 1.0722x over previous
import functools
import jax
import jax.numpy as jnp
import numpy as np
from jax.experimental import pallas as pl
from jax.experimental.pallas import tpu as pltpu

Q = 1024
K = 100000
DIM = 64
C = 32
K1 = 100
TOPK = 5

KPAD = 102400
BK = 2048
NBLK = KPAD // BK
NROW64 = 64
CW = 128


def _distsel_body(q_ref, k_ref, qsq_ref, ksq_ref, outd_ref, outi_ref, d64_ref,
                  d_scr):
    i = pl.program_id(0)
    q = q_ref[...]
    k = k_ref[...]
    lane = jax.lax.broadcasted_iota(jnp.int32, (Q, BK), 1)
    lane_c = jax.lax.broadcasted_iota(jnp.int32, (Q, CW), 1)
    qk = jax.lax.dot_general(q, k, (((1,), (1,)), ((), ())),
                             preferred_element_type=jnp.float32)
    d = qsq_ref[...] + ksq_ref[...] - 2.0 * qk
    d64_ref[...] = d[:NROW64, :]
    d_scr[...] = d

    @pl.when(i == 0)
    def _init():
        outd_ref[...] = jnp.full((Q, CW), jnp.inf, dtype=jnp.float32)
        outi_ref[...] = jnp.zeros((Q, CW), dtype=jnp.int32)

    M = jnp.where(lane_c < TOPK, outd_ref[...], jnp.float32(jnp.inf))
    Mi = jnp.where(lane_c < TOPK, outi_ref[...], jnp.int32(0))
    for t in range(TOPK):
        dd = d_scr[...]
        m = jnp.min(dd, axis=1, keepdims=True)
        a = jnp.min(jnp.where(dd == m, lane, jnp.int32(BK)), axis=1,
                    keepdims=True)
        d_scr[...] = jnp.where(lane == a, jnp.float32(jnp.inf), dd)
        M = jnp.where(lane_c == TOPK + t, m, M)
        Mi = jnp.where(lane_c == TOPK + t, a + i * BK, Mi)

    nd = jnp.full((Q, CW), jnp.inf, dtype=jnp.float32)
    ni = jnp.zeros((Q, CW), dtype=jnp.int32)
    for t in range(TOPK):
        m = jnp.min(M, axis=1, keepdims=True)
        a = jnp.min(jnp.where(M == m, lane_c, jnp.int32(CW)), axis=1,
                    keepdims=True)
        mid = jnp.min(jnp.where(lane_c == a, Mi, jnp.int32(2**30)), axis=1,
                      keepdims=True)
        M = jnp.where(lane_c == a, jnp.float32(jnp.inf), M)
        nd = jnp.where(lane_c == t, m, nd)
        ni = jnp.where(lane_c == t, mid, ni)
    outd_ref[...] = nd
    outi_ref[...] = ni


def _dist_select(queries, keys_pad, q_sq, k_sq_pad):
    return pl.pallas_call(
        _distsel_body,
        grid=(NBLK,),
        in_specs=[
            pl.BlockSpec((Q, DIM), lambda i: (0, 0)),
            pl.BlockSpec((BK, DIM), lambda i: (i, 0)),
            pl.BlockSpec((Q, 1), lambda i: (0, 0)),
            pl.BlockSpec((1, BK), lambda i: (0, i)),
        ],
        out_specs=[
            pl.BlockSpec((Q, CW), lambda i: (0, 0)),
            pl.BlockSpec((Q, CW), lambda i: (0, 0)),
            pl.BlockSpec((NROW64, BK), lambda i: (0, i)),
        ],
        out_shape=[
            jax.ShapeDtypeStruct((Q, CW), jnp.float32),
            jax.ShapeDtypeStruct((Q, CW), jnp.int32),
            jax.ShapeDtypeStruct((NROW64, KPAD), jnp.float32),
        ],
        scratch_shapes=[pltpu.VMEM((Q, BK), jnp.float32)],
    )(queries, keys_pad, q_sq, k_sq_pad)


def kernel(queries, keys, query_categories, key_categories, complex_weight, WU, aU):
    keys_pad = jnp.pad(keys, ((0, KPAD - K), (0, 0)))
    q_sq = jnp.sum(queries * queries, axis=1, keepdims=True)
    k_sq = jnp.sum(keys * keys, axis=1)
    k_sq_pad = jnp.pad(k_sq, (0, KPAD - K), constant_values=1e30)[None, :]
    outd, outi, d64 = _dist_select(queries, keys_pad, q_sq, k_sq_pad)
    D5 = outd[:, :TOPK]
    I5 = outi[:, :TOPK]

    negd, I100 = jax.lax.top_k(-d64, K1)

    simu_category = key_categories[I5].astype(bool)
    currentu_category = jnp.broadcast_to(
        query_categories[:, None, :].astype(bool), simu_category.shape)
    remain_category = jnp.logical_and(
        jnp.logical_xor(currentu_category, simu_category), simu_category)
    mask = 1e20 * (1.0 - jnp.any(remain_category, axis=2).astype(jnp.float32))
    Dmat = D5 + mask

    index = jnp.argsort(Dmat, axis=0)
    line_id = jnp.tile(jnp.arange(Q).reshape(-1, 1), (1, TOPK)) * TOPK
    flat_pos = (line_id + index).reshape(-1)
    new_I = I100.reshape(-1)[flat_pos].reshape(Q, TOPK)
    simu_embed = keys[new_I]

    x = jnp.fft.rfft(simu_embed, axis=1, norm='ortho')
    weight = complex_weight[..., 0] + 1j * complex_weight[..., 1]
    weight = weight[:, :TOPK // 2 + 1, :]
    x = x * weight
    simu_embed = jnp.fft.irfft(x, n=TOPK, axis=1, norm='ortho')
    Wsimu_embed = (simu_embed.reshape(-1, DIM) @ WU).reshape(Q, -1, DIM)
    Wu_embed = jnp.repeat((queries @ WU)[:, None, :], TOPK, axis=1)
    W_embed = jnp.concatenate([Wsimu_embed, Wu_embed], axis=2)
    Wa_embed = jax.nn.leaky_relu(jnp.einsum('bij,j->bi', W_embed, aU),
                                 negative_slope=0.1)
    alpha_embed = jax.nn.softmax(Wa_embed, axis=1)
    anchor_user_embed = jnp.sum(Wsimu_embed * alpha_embed[:, :, None], axis=1)
    return anchor_user_embed
